# baseline (speedup 1.0000x reference)
import jax
import jax.numpy as jnp
from jax.experimental import pallas as pl

PENALTY = 1.2
BV = 25088


def _damp_kernel(ids_ref, logits_ref, out_ref):
    S = ids_ref.shape[1]
    vb = pl.program_id(1)

    ids_local = ids_ref[0] - vb * BV
    vids = jax.lax.broadcasted_iota(jnp.int32, (S, BV), 1)
    oh = jnp.where(ids_local == vids, 1.0, 0.0)

    r = jax.lax.broadcasted_iota(jnp.int32, (S, S), 0)
    c = jax.lax.broadcasted_iota(jnp.int32, (S, S), 1)
    tril = jnp.where(c < r, 1.0, 0.0)

    counts = jax.lax.dot(tril, oh, preferred_element_type=jnp.float32)

    x = logits_ref[0]
    out_ref[0] = jnp.where(counts > 0.0, x * (1.0 / PENALTY), x)


@jax.jit
def kernel(logits, input_ids):
    B, S, V = logits.shape
    ids3 = input_ids.reshape(B, S, 1)
    return pl.pallas_call(
        _damp_kernel,
        grid=(B, pl.cdiv(V, BV)),
        in_specs=[
            pl.BlockSpec((1, S, 1), lambda b, v: (b, 0, 0)),
            pl.BlockSpec((1, S, BV), lambda b, v: (b, 0, v)),
        ],
        out_specs=pl.BlockSpec((1, S, BV), lambda b, v: (b, 0, v)),
        out_shape=jax.ShapeDtypeStruct((B, S, V), logits.dtype),
    )(ids3, logits)


# speedup vs baseline: 1.0509x; 1.0509x over previous
import jax
import jax.numpy as jnp
from jax.experimental import pallas as pl

PENALTY = 1.2
BV = 50048


def _damp_kernel(ids_ref, logits_ref, out_ref):
    S = ids_ref.shape[1]
    vb = pl.program_id(1)

    ids_local = ids_ref[0] - vb * BV
    vids = jax.lax.broadcasted_iota(jnp.int32, (S, BV), 1)
    oh = jnp.where(ids_local == vids, 1.0, 0.0).astype(jnp.bfloat16)

    r = jax.lax.broadcasted_iota(jnp.int32, (S, S), 0)
    c = jax.lax.broadcasted_iota(jnp.int32, (S, S), 1)
    tril = jnp.where(c < r, 1.0, 0.0).astype(jnp.bfloat16)

    counts = jax.lax.dot(tril, oh, preferred_element_type=jnp.float32)

    x = logits_ref[0]
    out_ref[0] = jnp.where(counts > 0.0, x * (1.0 / PENALTY), x)


@jax.jit
def kernel(logits, input_ids):
    B, S, V = logits.shape
    ids3 = input_ids.reshape(B, S, 1)
    return pl.pallas_call(
        _damp_kernel,
        grid=(B, pl.cdiv(V, BV)),
        in_specs=[
            pl.BlockSpec((1, S, 1), lambda b, v: (b, 0, 0)),
            pl.BlockSpec((1, S, BV), lambda b, v: (b, 0, v)),
        ],
        out_specs=pl.BlockSpec((1, S, BV), lambda b, v: (b, 0, v)),
        out_shape=jax.ShapeDtypeStruct((B, S, V), logits.dtype),
    )(ids3, logits)
